# Initial kernel scaffold; baseline (speedup 1.0000x reference)
import jax
import jax.numpy as jnp
from jax.experimental import pallas as pl


def kernel(x, grad_map):
    raise NotImplementedError("write your pallas kernel here")


# speedup vs baseline: 4.8750x; 4.8750x over previous
import math

import jax
import jax.numpy as jnp
import numpy as np
from jax.experimental import pallas as pl
from jax.experimental.pallas import tpu as pltpu

_C = 196
_H = 224
_W = 224
_NEIGH = 10
_M = 10.0
_MAX_ITER = 20
_M_S_SQ = (_M / math.sqrt(_H * _W / _C)) ** 2
_BIG = np.int32(2 ** 30)


def _host_constants():
    num_cols = int(math.sqrt(_C * _W / _H))
    num_rows = int(math.ceil(_C / num_cols))
    gy = _H / num_rows
    gx = _W / num_cols
    cents = []
    for i in range(num_rows):
        for j in range(num_cols):
            if len(cents) >= _C:
                break
            cents.append((int((i + 0.5) * gy), int((j + 0.5) * gx)))
        if len(cents) >= _C:
            break
    rows = []
    for (y, x) in cents:
        y0 = max(0, y - _NEIGH)
        y1 = min(_H, y + _NEIGH)
        x0 = max(0, x - _NEIGH)
        x1 = min(_W, x + _NEIGH)
        ys = min(y0, _H - 20)
        rows.append((y0, y1, x0, x1, ys, y, x, 0))
    return np.asarray(rows, dtype=np.int32)


_BOUNDS = _host_constants()


def _slic_body(bounds_ref, x_ref, g_ref, out_ref,
               dist_ref, occ_ref, yf_ref, xf_ref, yi_ref, xi_ref,
               cy_ref, cx_ref, col_ref):
    yi_ref[...] = jax.lax.broadcasted_iota(jnp.int32, (_H, _W), 0)
    xi_ref[...] = jax.lax.broadcasted_iota(jnp.int32, (_H, _W), 1)
    yf_ref[...] = yi_ref[...].astype(jnp.float32)
    xf_ref[...] = xi_ref[...].astype(jnp.float32)
    occ_ref[...] = jnp.zeros((_H, _W), jnp.float32)

    def step_a(c, _):
        y0 = bounds_ref[c, 0]
        y1 = bounds_ref[c, 1]
        x0 = bounds_ref[c, 2]
        x1 = bounds_ref[c, 3]
        g = g_ref[0, 0]
        yi = yi_ref[...]
        xi = xi_ref[...]
        mask = (yi >= y0) & (yi < y1) & (xi >= x0) & (xi < x1)
        mv = jnp.min(jnp.where(mask, g, jnp.inf))
        elig = mask & (g == mv) & (occ_ref[...] == 0.0)
        flat = yi * _W + xi
        idx = jnp.min(jnp.where(elig, flat, _BIG))
        found = idx < _BIG
        w = jnp.int32(_W)
        ny = jnp.where(found, jax.lax.div(idx, w), bounds_ref[c, 5])
        nx = jnp.where(found, jax.lax.rem(idx, w), bounds_ref[c, 6])
        cy_ref[c] = ny
        cx_ref[c] = nx
        occ_ref[...] = jnp.where((yi == ny) & (xi == nx) & found,
                                 1.0, occ_ref[...])
        return 0

    jax.lax.fori_loop(0, _C, step_a, 0)

    def col_init(c, _):
        pick = (yi_ref[...] == cy_ref[c]) & (xi_ref[...] == cx_ref[c])
        for ch in range(3):
            col_ref[c, ch] = jnp.sum(jnp.where(pick, x_ref[0, ch], 0.0))
        return 0

    jax.lax.fori_loop(0, _C, col_init, 0)

    def assign(c, _):
        cyf = cy_ref[c].astype(jnp.float32)
        cxf = cx_ref[c].astype(jnp.float32)
        dr = x_ref[0, 0] - col_ref[c, 0]
        dg = x_ref[0, 1] - col_ref[c, 1]
        db = x_ref[0, 2] - col_ref[c, 2]
        dy = yf_ref[...] - cyf
        dx = xf_ref[...] - cxf
        d = (dr * dr + dg * dg + db * db) + _M_S_SQ * (dy * dy + dx * dx)
        better = d < dist_ref[...]
        dist_ref[...] = jnp.where(better, d, dist_ref[...])
        out_ref[0] = jnp.where(better, c, out_ref[0])
        return 0

    def run_assign():
        dist_ref[...] = jnp.full((_H, _W), jnp.inf, jnp.float32)
        jax.lax.fori_loop(0, _C, assign, 0)

    def update(c, _):
        m = out_ref[0] == c
        zero = jnp.zeros((_H, _W), jnp.float32)
        cnt = jnp.sum(jnp.where(m, 1.0, 0.0))
        sy = jnp.sum(jnp.where(m, yf_ref[...], zero))
        sx = jnp.sum(jnp.where(m, xf_ref[...], zero))
        sr = jnp.sum(jnp.where(m, x_ref[0, 0], zero))
        sg = jnp.sum(jnp.where(m, x_ref[0, 1], zero))
        sb = jnp.sum(jnp.where(m, x_ref[0, 2], zero))
        nz = cnt > 0.0
        safe = jnp.where(nz, cnt, 1.0)

        def _round_half_even(q):
            qi = q.astype(jnp.int32)
            frac = q - qi.astype(jnp.float32)
            odd = jax.lax.rem(qi, jnp.int32(2)) == 1
            up = (frac > 0.5) | ((frac == 0.5) & odd)
            return qi + jnp.where(up, 1, 0).astype(jnp.int32)

        ny = jnp.clip(_round_half_even(sy / safe), 0, _H - 1)
        nx = jnp.clip(_round_half_even(sx / safe), 0, _W - 1)
        cy_ref[c] = jnp.where(nz, ny, cy_ref[c])
        cx_ref[c] = jnp.where(nz, nx, cx_ref[c])
        col_ref[c, 0] = jnp.where(nz, sr / safe, col_ref[c, 0])
        col_ref[c, 1] = jnp.where(nz, sg / safe, col_ref[c, 1])
        col_ref[c, 2] = jnp.where(nz, sb / safe, col_ref[c, 2])
        return 0

    def slic_iter(_it, carry):
        run_assign()
        jax.lax.fori_loop(0, _C, update, 0)
        return carry

    jax.lax.fori_loop(0, _MAX_ITER - 1, slic_iter, 0)
    run_assign()


def kernel(x, grad_map):
    b = x.shape[0]
    bounds = jnp.asarray(_BOUNDS)
    return pl.pallas_call(
        _slic_body,
        grid=(b,),
        in_specs=[
            pl.BlockSpec(memory_space=pltpu.SMEM),
            pl.BlockSpec((1, 3, _H, _W), lambda i: (i, 0, 0, 0)),
            pl.BlockSpec((1, 1, _H, _W), lambda i: (i, 0, 0, 0)),
        ],
        out_specs=pl.BlockSpec((1, _H, _W), lambda i: (i, 0, 0)),
        out_shape=jax.ShapeDtypeStruct((b, _H, _W), jnp.int32),
        scratch_shapes=[
            pltpu.VMEM((_H, _W), jnp.float32),
            pltpu.VMEM((_H, _W), jnp.float32),
            pltpu.VMEM((_H, _W), jnp.float32),
            pltpu.VMEM((_H, _W), jnp.float32),
            pltpu.VMEM((_H, _W), jnp.int32),
            pltpu.VMEM((_H, _W), jnp.int32),
            pltpu.SMEM((_C,), jnp.int32),
            pltpu.SMEM((_C,), jnp.int32),
            pltpu.SMEM((_C, 3), jnp.float32),
        ],
        compiler_params=pltpu.CompilerParams(
            dimension_semantics=("arbitrary",),
        ),
    )(bounds, x, grad_map)


# speedup vs baseline: 6.9561x; 1.4269x over previous
import math

import jax
import jax.numpy as jnp
import numpy as np
from jax.experimental import pallas as pl
from jax.experimental.pallas import tpu as pltpu

_C = 196
_H = 224
_W = 224
_NEIGH = 10
_M = 10.0
_MAX_ITER = 20
_M_S_SQ = (_M / math.sqrt(_H * _W / _C)) ** 2
_BIG = np.int32(2 ** 30)


def _host_constants():
    num_cols = int(math.sqrt(_C * _W / _H))
    num_rows = int(math.ceil(_C / num_cols))
    gy = _H / num_rows
    gx = _W / num_cols
    cents = []
    for i in range(num_rows):
        for j in range(num_cols):
            if len(cents) >= _C:
                break
            cents.append((int((i + 0.5) * gy), int((j + 0.5) * gx)))
        if len(cents) >= _C:
            break
    rows = []
    for (y, x) in cents:
        y0 = max(0, y - _NEIGH)
        y1 = min(_H, y + _NEIGH)
        x0 = max(0, x - _NEIGH)
        x1 = min(_W, x + _NEIGH)
        ys = min(y0, _H - 20)
        rows.append((y0, y1, x0, x1, ys, y, x, 0))
    return np.asarray(rows, dtype=np.int32)


_BOUNDS = _host_constants()


def _slic_body(bounds_ref, x_ref, g_ref, out_ref,
               dist_ref, occ_ref, yf_ref, xf_ref, yi_ref, xi_ref,
               cy_ref, cx_ref, col_ref):
    yi_ref[...] = jax.lax.broadcasted_iota(jnp.int32, (_H, _W), 0)
    xi_ref[...] = jax.lax.broadcasted_iota(jnp.int32, (_H, _W), 1)
    yf_ref[...] = yi_ref[...].astype(jnp.float32)
    xf_ref[...] = xi_ref[...].astype(jnp.float32)
    occ_ref[...] = jnp.zeros((_H, _W), jnp.float32)

    def step_a(c, _):
        y0 = bounds_ref[c, 0]
        y1 = bounds_ref[c, 1]
        x0 = bounds_ref[c, 2]
        x1 = bounds_ref[c, 3]
        g = g_ref[0, 0]
        yi = yi_ref[...]
        xi = xi_ref[...]
        mask = (yi >= y0) & (yi < y1) & (xi >= x0) & (xi < x1)
        mv = jnp.min(jnp.where(mask, g, jnp.inf))
        elig = mask & (g == mv) & (occ_ref[...] == 0.0)
        flat = yi * _W + xi
        idx = jnp.min(jnp.where(elig, flat, _BIG))
        found = idx < _BIG
        w = jnp.int32(_W)
        ny = jnp.where(found, jax.lax.div(idx, w), bounds_ref[c, 5])
        nx = jnp.where(found, jax.lax.rem(idx, w), bounds_ref[c, 6])
        cy_ref[c] = ny
        cx_ref[c] = nx
        occ_ref[...] = jnp.where((yi == ny) & (xi == nx) & found,
                                 1.0, occ_ref[...])
        return 0

    jax.lax.fori_loop(0, _C, step_a, 0, unroll=4)

    def col_init(c, _):
        pick = (yi_ref[...] == cy_ref[c]) & (xi_ref[...] == cx_ref[c])
        for ch in range(3):
            col_ref[c, ch] = jnp.sum(jnp.where(pick, x_ref[0, ch], 0.0))
        return 0

    jax.lax.fori_loop(0, _C, col_init, 0, unroll=4)

    def assign(c, _):
        cyf = cy_ref[c].astype(jnp.float32)
        cxf = cx_ref[c].astype(jnp.float32)
        dr = x_ref[0, 0] - col_ref[c, 0]
        dg = x_ref[0, 1] - col_ref[c, 1]
        db = x_ref[0, 2] - col_ref[c, 2]
        dy = yf_ref[...] - cyf
        dx = xf_ref[...] - cxf
        d = (dr * dr + dg * dg + db * db) + _M_S_SQ * (dy * dy + dx * dx)
        better = d < dist_ref[...]
        dist_ref[...] = jnp.where(better, d, dist_ref[...])
        out_ref[0] = jnp.where(better, c, out_ref[0])
        return 0

    def run_assign():
        dist_ref[...] = jnp.full((_H, _W), jnp.inf, jnp.float32)
        jax.lax.fori_loop(0, _C, assign, 0, unroll=8)

    def update(c, _):
        m = out_ref[0] == c
        zero = jnp.zeros((_H, _W), jnp.float32)
        cnt = jnp.sum(jnp.where(m, 1.0, 0.0))
        sy = jnp.sum(jnp.where(m, yf_ref[...], zero))
        sx = jnp.sum(jnp.where(m, xf_ref[...], zero))
        sr = jnp.sum(jnp.where(m, x_ref[0, 0], zero))
        sg = jnp.sum(jnp.where(m, x_ref[0, 1], zero))
        sb = jnp.sum(jnp.where(m, x_ref[0, 2], zero))
        nz = cnt > 0.0
        safe = jnp.where(nz, cnt, 1.0)

        def _round_half_even(q):
            qi = q.astype(jnp.int32)
            frac = q - qi.astype(jnp.float32)
            odd = jax.lax.rem(qi, jnp.int32(2)) == 1
            up = (frac > 0.5) | ((frac == 0.5) & odd)
            return qi + jnp.where(up, 1, 0).astype(jnp.int32)

        ny = jnp.clip(_round_half_even(sy / safe), 0, _H - 1)
        nx = jnp.clip(_round_half_even(sx / safe), 0, _W - 1)
        cy_ref[c] = jnp.where(nz, ny, cy_ref[c])
        cx_ref[c] = jnp.where(nz, nx, cx_ref[c])
        col_ref[c, 0] = jnp.where(nz, sr / safe, col_ref[c, 0])
        col_ref[c, 1] = jnp.where(nz, sg / safe, col_ref[c, 1])
        col_ref[c, 2] = jnp.where(nz, sb / safe, col_ref[c, 2])
        return 0

    def slic_iter(_it, carry):
        run_assign()
        jax.lax.fori_loop(0, _C, update, 0, unroll=8)
        return carry

    jax.lax.fori_loop(0, _MAX_ITER - 1, slic_iter, 0)
    run_assign()


def kernel(x, grad_map):
    b = x.shape[0]
    bounds = jnp.asarray(_BOUNDS)
    return pl.pallas_call(
        _slic_body,
        grid=(b,),
        in_specs=[
            pl.BlockSpec(memory_space=pltpu.SMEM),
            pl.BlockSpec((1, 3, _H, _W), lambda i: (i, 0, 0, 0)),
            pl.BlockSpec((1, 1, _H, _W), lambda i: (i, 0, 0, 0)),
        ],
        out_specs=pl.BlockSpec((1, _H, _W), lambda i: (i, 0, 0)),
        out_shape=jax.ShapeDtypeStruct((b, _H, _W), jnp.int32),
        scratch_shapes=[
            pltpu.VMEM((_H, _W), jnp.float32),
            pltpu.VMEM((_H, _W), jnp.float32),
            pltpu.VMEM((_H, _W), jnp.float32),
            pltpu.VMEM((_H, _W), jnp.float32),
            pltpu.VMEM((_H, _W), jnp.int32),
            pltpu.VMEM((_H, _W), jnp.int32),
            pltpu.SMEM((_C,), jnp.int32),
            pltpu.SMEM((_C,), jnp.int32),
            pltpu.SMEM((_C, 3), jnp.float32),
        ],
        compiler_params=pltpu.CompilerParams(
            dimension_semantics=("parallel",),
        ),
    )(bounds, x, grad_map)


# speedup vs baseline: 8.6864x; 1.2487x over previous
import math

import jax
import jax.numpy as jnp
import numpy as np
from jax.experimental import pallas as pl
from jax.experimental.pallas import tpu as pltpu

_C = 196
_H = 224
_W = 224
_NEIGH = 10
_M = 10.0
_MAX_ITER = 20
_M_S_SQ = (_M / math.sqrt(_H * _W / _C)) ** 2
_BIG = np.int32(2 ** 30)


def _host_constants():
    num_cols = int(math.sqrt(_C * _W / _H))
    num_rows = int(math.ceil(_C / num_cols))
    gy = _H / num_rows
    gx = _W / num_cols
    cents = []
    for i in range(num_rows):
        for j in range(num_cols):
            if len(cents) >= _C:
                break
            cents.append((int((i + 0.5) * gy), int((j + 0.5) * gx)))
        if len(cents) >= _C:
            break
    rows = []
    for (y, x) in cents:
        y0 = max(0, y - _NEIGH)
        y1 = min(_H, y + _NEIGH)
        x0 = max(0, x - _NEIGH)
        x1 = min(_W, x + _NEIGH)
        ys = min(y0, _H - 20)
        rows.append((y0, y1, x0, x1, ys, y, x, 0))
    return np.asarray(rows, dtype=np.int32)


_BOUNDS = _host_constants()


def _slic_body(bounds_ref, x_ref, g_ref, out_ref,
               dist_ref, occ_ref, yf_ref, xf_ref, yi_ref, xi_ref,
               cy_ref, cx_ref, col_ref):
    yi_ref[...] = jax.lax.broadcasted_iota(jnp.int32, (_H, _W), 0)
    xi_ref[...] = jax.lax.broadcasted_iota(jnp.int32, (_H, _W), 1)
    yf_ref[...] = yi_ref[...].astype(jnp.float32)
    xf_ref[...] = xi_ref[...].astype(jnp.float32)
    occ_ref[...] = jnp.zeros((_H, _W), jnp.float32)

    def step_a(c, _):
        y0 = bounds_ref[c, 0]
        y1 = bounds_ref[c, 1]
        x0 = bounds_ref[c, 2]
        x1 = bounds_ref[c, 3]
        g = g_ref[0, 0]
        yi = yi_ref[...]
        xi = xi_ref[...]
        mask = (yi >= y0) & (yi < y1) & (xi >= x0) & (xi < x1)
        mv = jnp.min(jnp.where(mask, g, jnp.inf))
        elig = mask & (g == mv) & (occ_ref[...] == 0.0)
        flat = yi * _W + xi
        idx = jnp.min(jnp.where(elig, flat, _BIG))
        found = idx < _BIG
        w = jnp.int32(_W)
        ny = jnp.where(found, jax.lax.div(idx, w), bounds_ref[c, 5])
        nx = jnp.where(found, jax.lax.rem(idx, w), bounds_ref[c, 6])
        cy_ref[c] = ny
        cx_ref[c] = nx
        occ_ref[...] = jnp.where((yi == ny) & (xi == nx) & found,
                                 1.0, occ_ref[...])
        return 0

    jax.lax.fori_loop(0, _C, step_a, 0, unroll=4)

    def col_init(c, _):
        pick = (yi_ref[...] == cy_ref[c]) & (xi_ref[...] == cx_ref[c])
        for ch in range(3):
            col_ref[c, ch] = jnp.sum(jnp.where(pick, x_ref[0, ch], 0.0))
        return 0

    jax.lax.fori_loop(0, _C, col_init, 0, unroll=4)

    ycol = jax.lax.broadcasted_iota(jnp.int32, (_H, 1), 0).astype(jnp.float32)
    xrow = jax.lax.broadcasted_iota(jnp.int32, (1, _W), 1).astype(jnp.float32)

    def assign(c, _):
        cyf = cy_ref[c].astype(jnp.float32)
        cxf = cx_ref[c].astype(jnp.float32)
        dr = x_ref[0, 0] - col_ref[c, 0]
        dg = x_ref[0, 1] - col_ref[c, 1]
        db = x_ref[0, 2] - col_ref[c, 2]
        dyv = ycol - cyf
        dxv = xrow - cxf
        d = (dr * dr + dg * dg + db * db) + _M_S_SQ * (dyv * dyv + dxv * dxv)
        better = d < dist_ref[...]
        dist_ref[...] = jnp.where(better, d, dist_ref[...])
        out_ref[0] = jnp.where(better, c, out_ref[0])
        return 0

    def run_assign():
        dist_ref[...] = jnp.full((_H, _W), jnp.inf, jnp.float32)
        jax.lax.fori_loop(0, _C, assign, 0, unroll=8)

    def update(c, _):
        m = out_ref[0] == c
        zero = jnp.zeros((_H, _W), jnp.float32)
        mf = jnp.where(m, 1.0, zero)
        rows = jnp.sum(mf, axis=1, keepdims=True)
        cols = jnp.sum(mf, axis=0, keepdims=True)
        cnt = jnp.sum(rows)
        sy = jnp.sum(rows * ycol)
        sx = jnp.sum(cols * xrow)
        sr = jnp.sum(jnp.where(m, x_ref[0, 0], zero))
        sg = jnp.sum(jnp.where(m, x_ref[0, 1], zero))
        sb = jnp.sum(jnp.where(m, x_ref[0, 2], zero))
        nz = cnt > 0.0
        safe = jnp.where(nz, cnt, 1.0)

        def _round_half_even(q):
            qi = q.astype(jnp.int32)
            frac = q - qi.astype(jnp.float32)
            odd = jax.lax.rem(qi, jnp.int32(2)) == 1
            up = (frac > 0.5) | ((frac == 0.5) & odd)
            return qi + jnp.where(up, 1, 0).astype(jnp.int32)

        ny = jnp.clip(_round_half_even(sy / safe), 0, _H - 1)
        nx = jnp.clip(_round_half_even(sx / safe), 0, _W - 1)
        cy_ref[c] = jnp.where(nz, ny, cy_ref[c])
        cx_ref[c] = jnp.where(nz, nx, cx_ref[c])
        col_ref[c, 0] = jnp.where(nz, sr / safe, col_ref[c, 0])
        col_ref[c, 1] = jnp.where(nz, sg / safe, col_ref[c, 1])
        col_ref[c, 2] = jnp.where(nz, sb / safe, col_ref[c, 2])
        return 0

    def slic_iter(_it, carry):
        run_assign()
        jax.lax.fori_loop(0, _C, update, 0, unroll=8)
        return carry

    jax.lax.fori_loop(0, _MAX_ITER - 1, slic_iter, 0)
    run_assign()


def kernel(x, grad_map):
    b = x.shape[0]
    bounds = jnp.asarray(_BOUNDS)
    return pl.pallas_call(
        _slic_body,
        grid=(b,),
        in_specs=[
            pl.BlockSpec(memory_space=pltpu.SMEM),
            pl.BlockSpec((1, 3, _H, _W), lambda i: (i, 0, 0, 0)),
            pl.BlockSpec((1, 1, _H, _W), lambda i: (i, 0, 0, 0)),
        ],
        out_specs=pl.BlockSpec((1, _H, _W), lambda i: (i, 0, 0)),
        out_shape=jax.ShapeDtypeStruct((b, _H, _W), jnp.int32),
        scratch_shapes=[
            pltpu.VMEM((_H, _W), jnp.float32),
            pltpu.VMEM((_H, _W), jnp.float32),
            pltpu.VMEM((_H, _W), jnp.float32),
            pltpu.VMEM((_H, _W), jnp.float32),
            pltpu.VMEM((_H, _W), jnp.int32),
            pltpu.VMEM((_H, _W), jnp.int32),
            pltpu.SMEM((_C,), jnp.int32),
            pltpu.SMEM((_C,), jnp.int32),
            pltpu.SMEM((_C, 3), jnp.float32),
        ],
        compiler_params=pltpu.CompilerParams(
            dimension_semantics=("parallel",),
        ),
    )(bounds, x, grad_map)
